# baseline (speedup 1.0000x reference)
import functools

import jax
import jax.numpy as jnp
import numpy as np
from jax import lax
from jax.experimental import pallas as pl
from jax.experimental.pallas import tpu as pltpu
from jax.experimental.pallas import tpu_sc as plsc

N = 10000
RPAD = 10112
NC, NS = 2, 16
EB = 96
STRIPE = RPAD // NS

_mesh = plsc.VectorSubcoreMesh(core_axis_name="c", subcore_axis_name="s",
                               num_cores=NC, num_subcores=NS)


def _deg_body(dst2, zeros_hbm, ones_hbm, out, dacc, didx, ones_v):
    c = lax.axis_index("c")
    s = lax.axis_index("s")
    pltpu.sync_copy(zeros_hbm, dacc.at[pl.ds(s * STRIPE, STRIPE)])
    pltpu.sync_copy(ones_hbm, ones_v)
    plsc.subcore_barrier()
    w = c * NS + s
    nb = dst2.shape[0] // (NC * NS)

    def body(b, carry):
        pltpu.sync_copy(dst2.at[w * nb + b], didx)
        pltpu.sync_copy(ones_v, dacc.at[didx], add=True)
        return carry

    lax.fori_loop(0, nb, body, 0)
    plsc.subcore_barrier()
    pltpu.sync_copy(dacc.at[pl.ds(s * STRIPE, STRIPE)],
                    out.at[c, pl.ds(s * STRIPE, STRIPE)])


def _make_deg_kernel():
    return functools.partial(
        pl.kernel,
        out_type=jax.ShapeDtypeStruct((NC, RPAD, 128), jnp.float32),
        mesh=_mesh,
        scratch_types=[
            pltpu.VMEM_SHARED((RPAD, 128), jnp.float32),
            pltpu.VMEM((EB,), jnp.int32),
            pltpu.VMEM((EB, 128), jnp.float32),
        ],
    )(_deg_body)


NBUF = 3


def _agg_body(npass, nslab, nb, ytab, src2, dst2, zeros_hbm, out, acc,
              sidx_all, didx, rows, g0, g1, g2, d0, d1, d2):
    gsems = (g0, g1, g2)
    dsems = (d0, d1, d2)
    c = lax.axis_index("c")
    s = lax.axis_index("s")
    ng = nb // NBUF
    t0 = pl.multiple_of(s * (nb * EB), 8)
    pltpu.sync_copy(src2.at[pl.ds(t0, nb * EB)], sidx_all)

    def _fire(b, k):
        off = pl.multiple_of(b * EB, EB)
        pltpu.async_copy(ytab.at[sidx_all.at[pl.ds(off, EB)]], rows.at[k],
                         gsems[k])
        pltpu.async_copy(dst2.at[s * nb + b], didx.at[k], dsems[k])

    for p in range(npass):
        slab = c * npass + p

        def tbody(r, carry):
            sl = pl.ds(pl.multiple_of(r * 16, 16), 16)
            if p == 0:
                sidx_all[sl] = sidx_all[sl] * nslab + slab
            else:
                sidx_all[sl] = sidx_all[sl] + 1
            return carry

        lax.fori_loop(0, nb * EB // 16, tbody, 0)
        pltpu.sync_copy(zeros_hbm, acc.at[pl.ds(s * STRIPE, STRIPE)])
        plsc.subcore_barrier()
        for k in range(NBUF):
            _fire(k, k)

        def gbody(g, carry):
            for k in range(NBUF):
                b = g * NBUF + k
                off = pl.multiple_of(b * EB, EB)
                pltpu.make_async_copy(ytab.at[sidx_all.at[pl.ds(off, EB)]],
                                      rows.at[k], gsems[k]).wait()
                pltpu.make_async_copy(dst2.at[s * nb + b], didx.at[k],
                                      dsems[k]).wait()
                pltpu.sync_copy(rows.at[k], acc.at[didx.at[k]], add=True)

                @pl.when(g < ng - 1)
                def _():
                    _fire(b + NBUF, k)
            return carry

        lax.fori_loop(0, ng, gbody, 0)
        plsc.subcore_barrier()
        pltpu.sync_copy(acc.at[pl.ds(s * STRIPE, STRIPE)],
                        out.at[slab, pl.ds(s * STRIPE, STRIPE)])
        if p + 1 < npass:
            plsc.subcore_barrier()


def _make_agg_kernel(npass, nb):
    nslab = NC * npass
    return functools.partial(
        pl.kernel,
        out_type=jax.ShapeDtypeStruct((nslab, RPAD, 128), jnp.float32),
        mesh=_mesh,
        scratch_types=[
            pltpu.VMEM_SHARED((RPAD, 128), jnp.float32),
            pltpu.VMEM((nb * EB,), jnp.int32),
            pltpu.VMEM((NBUF, EB), jnp.int32),
            pltpu.VMEM((NBUF, EB, 128), jnp.float32),
            pltpu.SemaphoreType.DMA,
            pltpu.SemaphoreType.DMA,
            pltpu.SemaphoreType.DMA,
            pltpu.SemaphoreType.DMA,
            pltpu.SemaphoreType.DMA,
            pltpu.SemaphoreType.DMA,
        ],
    )(functools.partial(_agg_body, npass, nslab, nb))


_BR = 1000


def _scale_body(p_ref, x_ref, y_ref, dis_ref):
    d = 1.0 + p_ref[0][:, 0:1] + p_ref[1][:, 0:1]
    iv = lax.rsqrt(d)
    dis_ref[...] = iv
    y_ref[...] = x_ref[...] * iv


def _scale_call(degp, x):
    return pl.pallas_call(
        _scale_body,
        grid=(N // _BR,),
        in_specs=[
            pl.BlockSpec((NC, _BR, 128), lambda i: (0, i, 0)),
            pl.BlockSpec((_BR, x.shape[1]), lambda i: (i, 0)),
        ],
        out_specs=[
            pl.BlockSpec((_BR, x.shape[1]), lambda i: (i, 0)),
            pl.BlockSpec((_BR, 1), lambda i: (i, 0)),
        ],
        out_shape=[
            jax.ShapeDtypeStruct((N, x.shape[1]), jnp.float32),
            jax.ShapeDtypeStruct((N, 1), jnp.float32),
        ],
    )(degp, x)


_BN_S = float(1.0 / np.sqrt(1.0 + 1e-5))


def _mm1_body(z_ref, y_ref, dis_ref, w_ref, b_ref, g_ref, be_ref, y2_ref):
    zcat = jnp.concatenate([z_ref[k] for k in range(2)], axis=1)
    u = (zcat + y_ref[...]) * dis_ref[...]
    h = jnp.dot(u, w_ref[...], preferred_element_type=jnp.float32) + b_ref[...]
    h = h * (g_ref[...] * _BN_S) + be_ref[...]
    h = jnp.maximum(h, 0.0)
    y2_ref[...] = h * dis_ref[...]


def _mm1_call(z1, y1, dis, W1, b1, g1, be1):
    return pl.pallas_call(
        _mm1_body,
        grid=(N // _BR,),
        in_specs=[
            pl.BlockSpec((2, _BR, 128), lambda i: (0, i, 0)),
            pl.BlockSpec((_BR, 256), lambda i: (i, 0)),
            pl.BlockSpec((_BR, 1), lambda i: (i, 0)),
            pl.BlockSpec((256, 512), lambda i: (0, 0)),
            pl.BlockSpec((1, 512), lambda i: (0, 0)),
            pl.BlockSpec((1, 512), lambda i: (0, 0)),
            pl.BlockSpec((1, 512), lambda i: (0, 0)),
        ],
        out_specs=pl.BlockSpec((_BR, 512), lambda i: (i, 0)),
        out_shape=jax.ShapeDtypeStruct((N, 512), jnp.float32),
    )(z1, y1, dis, W1, b1, g1, be1)


def _mm2_body(z_ref, y_ref, dis_ref, w_ref, b_ref, g_ref, be_ref, fw_ref, fb_ref,
              o_ref):
    zcat = jnp.concatenate([z_ref[k] for k in range(4)], axis=1)
    u = (zcat + y_ref[...]) * dis_ref[...]
    h = jnp.dot(u, w_ref[...], preferred_element_type=jnp.float32) + b_ref[...]
    h = h * (g_ref[...] * _BN_S) + be_ref[...]
    h = jnp.maximum(h, 0.0)
    o = jnp.dot(h, fw_ref[...], preferred_element_type=jnp.float32) + fb_ref[...]
    o_ref[...] = jax.nn.sigmoid(o)


def _mm2_call(z2, y2, dis, W2, b2, g2, be2, fcW, fcb):
    return pl.pallas_call(
        _mm2_body,
        grid=(N // _BR,),
        in_specs=[
            pl.BlockSpec((4, _BR, 128), lambda i: (0, i, 0)),
            pl.BlockSpec((_BR, 512), lambda i: (i, 0)),
            pl.BlockSpec((_BR, 1), lambda i: (i, 0)),
            pl.BlockSpec((512, 512), lambda i: (0, 0)),
            pl.BlockSpec((1, 512), lambda i: (0, 0)),
            pl.BlockSpec((1, 512), lambda i: (0, 0)),
            pl.BlockSpec((1, 512), lambda i: (0, 0)),
            pl.BlockSpec((512, 256), lambda i: (0, 0)),
            pl.BlockSpec((1, 256), lambda i: (0, 0)),
        ],
        out_specs=pl.BlockSpec((_BR, 256), lambda i: (i, 0)),
        out_shape=jax.ShapeDtypeStruct((N, 256), jnp.float32),
    )(z2, y2, dis, W2, b2, g2, be2, fcW, fcb)


def kernel(x, edge_index, W1, b1, gamma1, beta1, W2, b2, gamma2, beta2, fcW, fcb):
    E = edge_index.shape[1]
    quantum = EB * NS * NC * NBUF
    epad = ((E + quantum - 1) // quantum) * quantum
    src = edge_index[0].astype(jnp.int32)
    dst = edge_index[1].astype(jnp.int32)
    src2 = jnp.concatenate([src, jnp.zeros((epad - E,), jnp.int32)]).reshape(-1, EB)
    dst2 = jnp.concatenate([dst, jnp.full((epad - E,), N, jnp.int32)]).reshape(-1, EB)

    zeros_w = jnp.zeros((STRIPE, 128), jnp.float32)
    ones_w = jnp.ones((EB, 128), jnp.float32)

    nb = epad // EB // NS
    src1 = src2.reshape(-1)
    degp = _make_deg_kernel()(dst2, zeros_w, ones_w)
    y1, dis = _scale_call(degp, x)
    z1 = _make_agg_kernel(1, nb)(y1.reshape(2 * N, 128), src1, dst2, zeros_w)
    y2 = _mm1_call(z1, y1, dis, W1, b1.reshape(1, -1), gamma1.reshape(1, -1),
                   beta1.reshape(1, -1))
    z2 = _make_agg_kernel(2, nb)(y2.reshape(4 * N, 128), src1, dst2, zeros_w)
    out = _mm2_call(z2, y2, dis, W2, b2.reshape(1, -1), gamma2.reshape(1, -1),
                    beta2.reshape(1, -1), fcW, fcb.reshape(1, -1))
    return out


# speedup vs baseline: 1.8714x; 1.8714x over previous
import functools

import jax
import jax.numpy as jnp
import numpy as np
from jax import lax
from jax.experimental import pallas as pl
from jax.experimental.pallas import tpu as pltpu
from jax.experimental.pallas import tpu_sc as plsc

N = 10000
RPAD = 10240
NC, NS = 2, 16
EB = 112
STRIPE = RPAD // NS

_mesh = plsc.VectorSubcoreMesh(core_axis_name="c", subcore_axis_name="s",
                               num_cores=NC, num_subcores=NS)


def _deg_body(dst2, zeros_hbm, ones_hbm, out, dacc, didx, ones_v):
    c = lax.axis_index("c")
    s = lax.axis_index("s")
    pltpu.sync_copy(zeros_hbm, dacc.at[pl.ds(s * STRIPE, STRIPE)])
    pltpu.sync_copy(ones_hbm, ones_v)
    plsc.subcore_barrier()
    w = c * NS + s
    nb = dst2.shape[0] // (NC * NS)

    def body(b, carry):
        pltpu.sync_copy(dst2.at[w * nb + b], didx)
        pltpu.sync_copy(ones_v, dacc.at[didx], add=True)
        return carry

    lax.fori_loop(0, nb, body, 0)
    plsc.subcore_barrier()
    pltpu.sync_copy(dacc.at[pl.ds(s * STRIPE, STRIPE)],
                    out.at[c, pl.ds(s * STRIPE, STRIPE)])


def _make_deg_kernel():
    return functools.partial(
        pl.kernel,
        out_type=jax.ShapeDtypeStruct((NC, RPAD, 128), jnp.float32),
        mesh=_mesh,
        scratch_types=[
            pltpu.VMEM_SHARED((RPAD, 128), jnp.float32),
            pltpu.VMEM((EB,), jnp.int32),
            pltpu.VMEM((EB, 128), jnp.float32),
        ],
    )(_deg_body)


NBUF = 3


def _agg_body(npass, nslab, nb, ytab, src2, dst2, zeros_hbm, out, acc,
              sidx, didx, rows, s0, s1, s2, d0, d1, d2):
    sems = (s0, s1, s2)
    dsems = (d0, d1, d2)
    c = lax.axis_index("c")
    s = lax.axis_index("s")
    ng = nb // NBUF

    def _load_and_fire(b, k, slab):
        pltpu.async_copy(dst2.at[s * nb + b], didx.at[k], dsems[k])
        pltpu.sync_copy(src2.at[s * nb + b], sidx.at[k])
        for j in range(EB // 16):
            sl = (k, pl.ds(j * 16, 16))
            sidx[sl] = sidx[sl] * nslab + slab
        pltpu.async_copy(ytab.at[sidx.at[k]], rows.at[k], sems[k])

    for p in range(npass):
        slab = c * npass + p
        pltpu.sync_copy(zeros_hbm, acc.at[pl.ds(s * STRIPE, STRIPE)])
        plsc.subcore_barrier()
        for k in range(NBUF):
            _load_and_fire(k, k, slab)

        def gbody(g, carry):
            for k in range(NBUF):
                b = g * NBUF + k
                pltpu.make_async_copy(ytab.at[sidx.at[k]], rows.at[k],
                                      sems[k]).wait()
                pltpu.make_async_copy(dst2.at[s * nb + b], didx.at[k],
                                      dsems[k]).wait()
                pltpu.sync_copy(rows.at[k], acc.at[didx.at[k]], add=True)

                @pl.when(g < ng - 1)
                def _():
                    _load_and_fire(b + NBUF, k, slab)
            return carry

        lax.fori_loop(0, ng, gbody, 0)
        plsc.subcore_barrier()
        pltpu.sync_copy(acc.at[pl.ds(s * STRIPE, STRIPE)],
                        out.at[slab, pl.ds(s * STRIPE, STRIPE)])
        if p + 1 < npass:
            plsc.subcore_barrier()


def _make_agg_kernel(npass, nb):
    nslab = NC * npass
    return functools.partial(
        pl.kernel,
        out_type=jax.ShapeDtypeStruct((nslab, RPAD, 128), jnp.float32),
        mesh=_mesh,
        scratch_types=[
            pltpu.VMEM_SHARED((RPAD, 128), jnp.float32),
            pltpu.VMEM((NBUF, EB), jnp.int32),
            pltpu.VMEM((NBUF, EB), jnp.int32),
            pltpu.VMEM((NBUF, EB, 128), jnp.float32),
            pltpu.SemaphoreType.DMA,
            pltpu.SemaphoreType.DMA,
            pltpu.SemaphoreType.DMA,
            pltpu.SemaphoreType.DMA,
            pltpu.SemaphoreType.DMA,
            pltpu.SemaphoreType.DMA,
        ],
    )(functools.partial(_agg_body, npass, nslab, nb))


_BR = 1000


def _scale_body(p_ref, x_ref, y_ref, dis_ref):
    d = 1.0 + p_ref[0][:, 0:1] + p_ref[1][:, 0:1]
    iv = lax.rsqrt(d)
    dis_ref[...] = iv
    y_ref[...] = x_ref[...] * iv


def _scale_call(degp, x):
    return pl.pallas_call(
        _scale_body,
        grid=(N // _BR,),
        in_specs=[
            pl.BlockSpec((NC, _BR, 128), lambda i: (0, i, 0)),
            pl.BlockSpec((_BR, x.shape[1]), lambda i: (i, 0)),
        ],
        out_specs=[
            pl.BlockSpec((_BR, x.shape[1]), lambda i: (i, 0)),
            pl.BlockSpec((_BR, 1), lambda i: (i, 0)),
        ],
        out_shape=[
            jax.ShapeDtypeStruct((N, x.shape[1]), jnp.float32),
            jax.ShapeDtypeStruct((N, 1), jnp.float32),
        ],
    )(degp, x)


_BN_S = float(1.0 / np.sqrt(1.0 + 1e-5))


def _mm1_body(z_ref, y_ref, dis_ref, w_ref, b_ref, g_ref, be_ref, y2_ref):
    zcat = jnp.concatenate([z_ref[k] for k in range(2)], axis=1)
    u = (zcat + y_ref[...]) * dis_ref[...]
    h = jnp.dot(u, w_ref[...], preferred_element_type=jnp.float32) + b_ref[...]
    h = h * (g_ref[...] * _BN_S) + be_ref[...]
    h = jnp.maximum(h, 0.0)
    y2_ref[...] = h * dis_ref[...]


def _mm1_call(z1, y1, dis, W1, b1, g1, be1):
    return pl.pallas_call(
        _mm1_body,
        grid=(N // _BR,),
        in_specs=[
            pl.BlockSpec((2, _BR, 128), lambda i: (0, i, 0)),
            pl.BlockSpec((_BR, 256), lambda i: (i, 0)),
            pl.BlockSpec((_BR, 1), lambda i: (i, 0)),
            pl.BlockSpec((256, 512), lambda i: (0, 0)),
            pl.BlockSpec((1, 512), lambda i: (0, 0)),
            pl.BlockSpec((1, 512), lambda i: (0, 0)),
            pl.BlockSpec((1, 512), lambda i: (0, 0)),
        ],
        out_specs=pl.BlockSpec((_BR, 512), lambda i: (i, 0)),
        out_shape=jax.ShapeDtypeStruct((N, 512), jnp.float32),
    )(z1, y1, dis, W1, b1, g1, be1)


def _mm2_body(z_ref, y_ref, dis_ref, w_ref, b_ref, g_ref, be_ref, fw_ref, fb_ref,
              o_ref):
    zcat = jnp.concatenate([z_ref[k] for k in range(4)], axis=1)
    u = (zcat + y_ref[...]) * dis_ref[...]
    h = jnp.dot(u, w_ref[...], preferred_element_type=jnp.float32) + b_ref[...]
    h = h * (g_ref[...] * _BN_S) + be_ref[...]
    h = jnp.maximum(h, 0.0)
    o = jnp.dot(h, fw_ref[...], preferred_element_type=jnp.float32) + fb_ref[...]
    o_ref[...] = jax.nn.sigmoid(o)


def _mm2_call(z2, y2, dis, W2, b2, g2, be2, fcW, fcb):
    return pl.pallas_call(
        _mm2_body,
        grid=(N // _BR,),
        in_specs=[
            pl.BlockSpec((4, _BR, 128), lambda i: (0, i, 0)),
            pl.BlockSpec((_BR, 512), lambda i: (i, 0)),
            pl.BlockSpec((_BR, 1), lambda i: (i, 0)),
            pl.BlockSpec((512, 512), lambda i: (0, 0)),
            pl.BlockSpec((1, 512), lambda i: (0, 0)),
            pl.BlockSpec((1, 512), lambda i: (0, 0)),
            pl.BlockSpec((1, 512), lambda i: (0, 0)),
            pl.BlockSpec((512, 256), lambda i: (0, 0)),
            pl.BlockSpec((1, 256), lambda i: (0, 0)),
        ],
        out_specs=pl.BlockSpec((_BR, 256), lambda i: (i, 0)),
        out_shape=jax.ShapeDtypeStruct((N, 256), jnp.float32),
    )(z2, y2, dis, W2, b2, g2, be2, fcW, fcb)


def kernel(x, edge_index, W1, b1, gamma1, beta1, W2, b2, gamma2, beta2, fcW, fcb):
    E = edge_index.shape[1]
    quantum = EB * NS * NC * NBUF
    epad = ((E + quantum - 1) // quantum) * quantum
    src = edge_index[0].astype(jnp.int32)
    dst = edge_index[1].astype(jnp.int32)
    src2 = jnp.concatenate([src, jnp.zeros((epad - E,), jnp.int32)]).reshape(-1, EB)
    dst2 = jnp.concatenate([dst, jnp.full((epad - E,), N, jnp.int32)]).reshape(-1, EB)

    zeros_w = jnp.zeros((STRIPE, 128), jnp.float32)
    ones_w = jnp.ones((EB, 128), jnp.float32)

    nb = epad // EB // NS
    degp = _make_deg_kernel()(dst2, zeros_w, ones_w)
    y1, dis = _scale_call(degp, x)
    z1 = _make_agg_kernel(1, nb)(y1.reshape(2 * N, 128), src2, dst2, zeros_w)
    y2 = _mm1_call(z1, y1, dis, W1, b1.reshape(1, -1), gamma1.reshape(1, -1),
                   beta1.reshape(1, -1))
    z2 = _make_agg_kernel(2, nb)(y2.reshape(4 * N, 128), src2, dst2, zeros_w)
    out = _mm2_call(z2, y2, dis, W2, b2.reshape(1, -1), gamma2.reshape(1, -1),
                    beta2.reshape(1, -1), fcW, fcb.reshape(1, -1))
    return out


# speedup vs baseline: 1.9245x; 1.0283x over previous
import functools

import jax
import jax.numpy as jnp
import numpy as np
from jax import lax
from jax.experimental import pallas as pl
from jax.experimental.pallas import tpu as pltpu
from jax.experimental.pallas import tpu_sc as plsc

N = 10000
RPAD = 10240
NC, NS = 2, 16
EB = 112
STRIPE = RPAD // NS

_mesh = plsc.VectorSubcoreMesh(core_axis_name="c", subcore_axis_name="s",
                               num_cores=NC, num_subcores=NS)


def _deg_body(dst2, zeros_hbm, ones_hbm, out, dacc, didx, ones_v, d0, d1, d2):
    dsems = (d0, d1, d2)
    c = lax.axis_index("c")
    s = lax.axis_index("s")
    pltpu.sync_copy(zeros_hbm, dacc.at[pl.ds(s * STRIPE, STRIPE)])
    pltpu.sync_copy(ones_hbm, ones_v)
    plsc.subcore_barrier()
    w = c * NS + s
    nb = dst2.shape[0] // (NC * NS)
    ng = nb // NBUF
    for k in range(NBUF):
        pltpu.async_copy(dst2.at[w * nb + k], didx.at[k], dsems[k])

    def body(g, carry):
        for k in range(NBUF):
            b = g * NBUF + k
            pltpu.make_async_copy(dst2.at[w * nb + b], didx.at[k],
                                  dsems[k]).wait()
            pltpu.sync_copy(ones_v, dacc.at[didx.at[k]], add=True)

            @pl.when(g < ng - 1)
            def _():
                pltpu.async_copy(dst2.at[w * nb + b + NBUF], didx.at[k],
                                 dsems[k])
        return carry

    lax.fori_loop(0, ng, body, 0)
    plsc.subcore_barrier()
    pltpu.sync_copy(dacc.at[pl.ds(s * STRIPE, STRIPE)],
                    out.at[c, pl.ds(s * STRIPE, STRIPE)])


def _make_deg_kernel():
    return functools.partial(
        pl.kernel,
        out_type=jax.ShapeDtypeStruct((NC, RPAD, 128), jnp.float32),
        mesh=_mesh,
        scratch_types=[
            pltpu.VMEM_SHARED((RPAD, 128), jnp.float32),
            pltpu.VMEM((NBUF, EB), jnp.int32),
            pltpu.VMEM((EB, 128), jnp.float32),
            pltpu.SemaphoreType.DMA,
            pltpu.SemaphoreType.DMA,
            pltpu.SemaphoreType.DMA,
        ],
    )(_deg_body)


NBUF = 3


def _agg_body(npass, nslab, nb, ytab, src2, dst2, zeros_hbm, out, acc,
              sidx, didx, rows, s0, s1, s2, d0, d1, d2):
    sems = (s0, s1, s2)
    dsems = (d0, d1, d2)
    c = lax.axis_index("c")
    s = lax.axis_index("s")
    ng = nb // NBUF

    def _load_and_fire(b, k, slab):
        pltpu.async_copy(dst2.at[s * nb + b], didx.at[k], dsems[k])
        pltpu.sync_copy(src2.at[s * nb + b], sidx.at[k])
        for j in range(EB // 16):
            sl = (k, pl.ds(j * 16, 16))
            sidx[sl] = sidx[sl] * nslab + slab
        pltpu.async_copy(ytab.at[sidx.at[k]], rows.at[k], sems[k])

    for p in range(npass):
        slab = c * npass + p
        pltpu.sync_copy(zeros_hbm, acc.at[pl.ds(s * STRIPE, STRIPE)])
        plsc.subcore_barrier()
        for k in range(NBUF):
            _load_and_fire(k, k, slab)

        def gbody(g, carry):
            for k in range(NBUF):
                b = g * NBUF + k
                pltpu.make_async_copy(ytab.at[sidx.at[k]], rows.at[k],
                                      sems[k]).wait()
                pltpu.make_async_copy(dst2.at[s * nb + b], didx.at[k],
                                      dsems[k]).wait()
                pltpu.sync_copy(rows.at[k], acc.at[didx.at[k]], add=True)

                @pl.when(g < ng - 1)
                def _():
                    _load_and_fire(b + NBUF, k, slab)
            return carry

        lax.fori_loop(0, ng, gbody, 0)
        plsc.subcore_barrier()
        pltpu.sync_copy(acc.at[pl.ds(s * STRIPE, STRIPE)],
                        out.at[slab, pl.ds(s * STRIPE, STRIPE)])
        if p + 1 < npass:
            plsc.subcore_barrier()


def _make_agg_kernel(npass, nb):
    nslab = NC * npass
    return functools.partial(
        pl.kernel,
        out_type=jax.ShapeDtypeStruct((nslab, RPAD, 128), jnp.float32),
        mesh=_mesh,
        scratch_types=[
            pltpu.VMEM_SHARED((RPAD, 128), jnp.float32),
            pltpu.VMEM((NBUF, EB), jnp.int32),
            pltpu.VMEM((NBUF, EB), jnp.int32),
            pltpu.VMEM((NBUF, EB, 128), jnp.float32),
            pltpu.SemaphoreType.DMA,
            pltpu.SemaphoreType.DMA,
            pltpu.SemaphoreType.DMA,
            pltpu.SemaphoreType.DMA,
            pltpu.SemaphoreType.DMA,
            pltpu.SemaphoreType.DMA,
        ],
    )(functools.partial(_agg_body, npass, nslab, nb))


_BR = 1000


def _scale_body(p_ref, x_ref, y_ref, dis_ref):
    d = 1.0 + p_ref[0][:, 0:1] + p_ref[1][:, 0:1]
    iv = lax.rsqrt(d)
    dis_ref[...] = iv
    y_ref[...] = x_ref[...] * iv


def _scale_call(degp, x):
    return pl.pallas_call(
        _scale_body,
        grid=(N // _BR,),
        in_specs=[
            pl.BlockSpec((NC, _BR, 128), lambda i: (0, i, 0)),
            pl.BlockSpec((_BR, x.shape[1]), lambda i: (i, 0)),
        ],
        out_specs=[
            pl.BlockSpec((_BR, x.shape[1]), lambda i: (i, 0)),
            pl.BlockSpec((_BR, 1), lambda i: (i, 0)),
        ],
        out_shape=[
            jax.ShapeDtypeStruct((N, x.shape[1]), jnp.float32),
            jax.ShapeDtypeStruct((N, 1), jnp.float32),
        ],
    )(degp, x)


_BN_S = float(1.0 / np.sqrt(1.0 + 1e-5))


def _mm1_body(z_ref, y_ref, dis_ref, w_ref, b_ref, g_ref, be_ref, y2_ref):
    zcat = jnp.concatenate([z_ref[k] for k in range(2)], axis=1)
    u = (zcat + y_ref[...]) * dis_ref[...]
    h = jnp.dot(u, w_ref[...], preferred_element_type=jnp.float32) + b_ref[...]
    h = h * (g_ref[...] * _BN_S) + be_ref[...]
    h = jnp.maximum(h, 0.0)
    y2_ref[...] = h * dis_ref[...]


def _mm1_call(z1, y1, dis, W1, b1, g1, be1):
    return pl.pallas_call(
        _mm1_body,
        grid=(N // _BR,),
        in_specs=[
            pl.BlockSpec((2, _BR, 128), lambda i: (0, i, 0)),
            pl.BlockSpec((_BR, 256), lambda i: (i, 0)),
            pl.BlockSpec((_BR, 1), lambda i: (i, 0)),
            pl.BlockSpec((256, 512), lambda i: (0, 0)),
            pl.BlockSpec((1, 512), lambda i: (0, 0)),
            pl.BlockSpec((1, 512), lambda i: (0, 0)),
            pl.BlockSpec((1, 512), lambda i: (0, 0)),
        ],
        out_specs=pl.BlockSpec((_BR, 512), lambda i: (i, 0)),
        out_shape=jax.ShapeDtypeStruct((N, 512), jnp.float32),
    )(z1, y1, dis, W1, b1, g1, be1)


def _mm2_body(z_ref, y_ref, dis_ref, w_ref, b_ref, g_ref, be_ref, fw_ref, fb_ref,
              o_ref):
    zcat = jnp.concatenate([z_ref[k] for k in range(4)], axis=1)
    u = (zcat + y_ref[...]) * dis_ref[...]
    h = jnp.dot(u, w_ref[...], preferred_element_type=jnp.float32) + b_ref[...]
    h = h * (g_ref[...] * _BN_S) + be_ref[...]
    h = jnp.maximum(h, 0.0)
    o = jnp.dot(h, fw_ref[...], preferred_element_type=jnp.float32) + fb_ref[...]
    o_ref[...] = jax.nn.sigmoid(o)


def _mm2_call(z2, y2, dis, W2, b2, g2, be2, fcW, fcb):
    return pl.pallas_call(
        _mm2_body,
        grid=(N // _BR,),
        in_specs=[
            pl.BlockSpec((4, _BR, 128), lambda i: (0, i, 0)),
            pl.BlockSpec((_BR, 512), lambda i: (i, 0)),
            pl.BlockSpec((_BR, 1), lambda i: (i, 0)),
            pl.BlockSpec((512, 512), lambda i: (0, 0)),
            pl.BlockSpec((1, 512), lambda i: (0, 0)),
            pl.BlockSpec((1, 512), lambda i: (0, 0)),
            pl.BlockSpec((1, 512), lambda i: (0, 0)),
            pl.BlockSpec((512, 256), lambda i: (0, 0)),
            pl.BlockSpec((1, 256), lambda i: (0, 0)),
        ],
        out_specs=pl.BlockSpec((_BR, 256), lambda i: (i, 0)),
        out_shape=jax.ShapeDtypeStruct((N, 256), jnp.float32),
    )(z2, y2, dis, W2, b2, g2, be2, fcW, fcb)


def kernel(x, edge_index, W1, b1, gamma1, beta1, W2, b2, gamma2, beta2, fcW, fcb):
    E = edge_index.shape[1]
    quantum = EB * NS * NC * NBUF
    epad = ((E + quantum - 1) // quantum) * quantum
    src = edge_index[0].astype(jnp.int32)
    dst = edge_index[1].astype(jnp.int32)
    src2 = jnp.concatenate([src, jnp.zeros((epad - E,), jnp.int32)]).reshape(-1, EB)
    dst2 = jnp.concatenate([dst, jnp.full((epad - E,), N, jnp.int32)]).reshape(-1, EB)

    zeros_w = jnp.zeros((STRIPE, 128), jnp.float32)
    ones_w = jnp.ones((EB, 128), jnp.float32)

    nb = epad // EB // NS
    degp = _make_deg_kernel()(dst2, zeros_w, ones_w)
    y1, dis = _scale_call(degp, x)
    z1 = _make_agg_kernel(1, nb)(y1.reshape(2 * N, 128), src2, dst2, zeros_w)
    y2 = _mm1_call(z1, y1, dis, W1, b1.reshape(1, -1), gamma1.reshape(1, -1),
                   beta1.reshape(1, -1))
    z2 = _make_agg_kernel(2, nb)(y2.reshape(4 * N, 128), src2, dst2, zeros_w)
    out = _mm2_call(z2, y2, dis, W2, b2.reshape(1, -1), gamma2.reshape(1, -1),
                    beta2.reshape(1, -1), fcW, fcb.reshape(1, -1))
    return out


# speedup vs baseline: 2.0918x; 1.0870x over previous
import functools

import jax
import jax.numpy as jnp
import numpy as np
from jax import lax
from jax.experimental import pallas as pl
from jax.experimental.pallas import tpu as pltpu
from jax.experimental.pallas import tpu_sc as plsc

N = 10000
RPAD = 10240
NC, NS = 2, 16
EB = 112
STRIPE = RPAD // NS

_mesh = plsc.VectorSubcoreMesh(core_axis_name="c", subcore_axis_name="s",
                               num_cores=NC, num_subcores=NS)


def _deg_body(dst2, zeros_hbm, ones_hbm, out, dacc, didx, ones_v, d0, d1, d2):
    dsems = (d0, d1, d2)
    c = lax.axis_index("c")
    s = lax.axis_index("s")
    pltpu.sync_copy(zeros_hbm, dacc.at[pl.ds(s * STRIPE, STRIPE)])
    pltpu.sync_copy(ones_hbm, ones_v)
    plsc.subcore_barrier()
    w = c * NS + s
    nb = dst2.shape[0] // (NC * NS)
    ng = nb // NBUF
    for k in range(NBUF):
        pltpu.async_copy(dst2.at[w * nb + k], didx.at[k], dsems[k])

    def body(g, carry):
        for k in range(NBUF):
            b = g * NBUF + k
            pltpu.make_async_copy(dst2.at[w * nb + b], didx.at[k],
                                  dsems[k]).wait()
            pltpu.sync_copy(ones_v, dacc.at[didx.at[k]], add=True)

            @pl.when(g < ng - 1)
            def _():
                pltpu.async_copy(dst2.at[w * nb + b + NBUF], didx.at[k],
                                 dsems[k])
        return carry

    lax.fori_loop(0, ng, body, 0)
    plsc.subcore_barrier()
    pltpu.sync_copy(dacc.at[pl.ds(s * STRIPE, STRIPE)],
                    out.at[c, pl.ds(s * STRIPE, STRIPE)])


def _make_deg_kernel():
    return functools.partial(
        pl.kernel,
        out_type=jax.ShapeDtypeStruct((NC, RPAD, 128), jnp.float32),
        mesh=_mesh,
        scratch_types=[
            pltpu.VMEM_SHARED((RPAD, 128), jnp.float32),
            pltpu.VMEM((NBUF, EB), jnp.int32),
            pltpu.VMEM((EB, 128), jnp.float32),
            pltpu.SemaphoreType.DMA,
            pltpu.SemaphoreType.DMA,
            pltpu.SemaphoreType.DMA,
        ],
    )(_deg_body)


NBUF = 3


def _agg_body(npass, nslab, nb, ytab, src2, dst2, zeros_hbm, out, acc,
              sidx, didx, rows, s0, s1, s2, d0, d1, d2, t0, t1, t2):
    sems = (s0, s1, s2)
    dsems = (d0, d1, d2)
    tsems = (t0, t1, t2)
    c = lax.axis_index("c")
    s = lax.axis_index("s")
    ng = nb // NBUF

    def _transform_and_fire(b, k, slab):
        for j in range(EB // 16):
            sl = (k, pl.ds(j * 16, 16))
            sidx[sl] = sidx[sl] * nslab + slab
        pltpu.async_copy(dst2.at[s * nb + b], didx.at[k], dsems[k])
        pltpu.async_copy(ytab.at[sidx.at[k]], rows.at[k], sems[k])

    for p in range(npass):
        slab = c * npass + p
        pltpu.sync_copy(zeros_hbm, acc.at[pl.ds(s * STRIPE, STRIPE)])
        plsc.subcore_barrier()
        for k in range(NBUF):
            pltpu.sync_copy(src2.at[s * nb + k], sidx.at[k])
            _transform_and_fire(k, k, slab)

        def gbody(g, carry):
            for k in range(NBUF):
                b = g * NBUF + k
                pltpu.make_async_copy(ytab.at[sidx.at[k]], rows.at[k],
                                      sems[k]).wait()

                @pl.when(g < ng - 1)
                def _():
                    pltpu.async_copy(src2.at[s * nb + b + NBUF], sidx.at[k],
                                     tsems[k])

                pltpu.make_async_copy(dst2.at[s * nb + b], didx.at[k],
                                      dsems[k]).wait()
                pltpu.sync_copy(rows.at[k], acc.at[didx.at[k]], add=True)

                @pl.when(g < ng - 1)
                def _():
                    pltpu.make_async_copy(src2.at[s * nb + b + NBUF],
                                          sidx.at[k], tsems[k]).wait()
                    _transform_and_fire(b + NBUF, k, slab)
            return carry

        lax.fori_loop(0, ng, gbody, 0)
        plsc.subcore_barrier()
        pltpu.sync_copy(acc.at[pl.ds(s * STRIPE, STRIPE)],
                        out.at[slab, pl.ds(s * STRIPE, STRIPE)])
        if p + 1 < npass:
            plsc.subcore_barrier()


def _make_agg_kernel(npass, nb):
    nslab = NC * npass
    return functools.partial(
        pl.kernel,
        out_type=jax.ShapeDtypeStruct((nslab, RPAD, 128), jnp.float32),
        mesh=_mesh,
        scratch_types=[
            pltpu.VMEM_SHARED((RPAD, 128), jnp.float32),
            pltpu.VMEM((NBUF, EB), jnp.int32),
            pltpu.VMEM((NBUF, EB), jnp.int32),
            pltpu.VMEM((NBUF, EB, 128), jnp.float32),
            pltpu.SemaphoreType.DMA,
            pltpu.SemaphoreType.DMA,
            pltpu.SemaphoreType.DMA,
            pltpu.SemaphoreType.DMA,
            pltpu.SemaphoreType.DMA,
            pltpu.SemaphoreType.DMA,
            pltpu.SemaphoreType.DMA,
            pltpu.SemaphoreType.DMA,
            pltpu.SemaphoreType.DMA,
        ],
    )(functools.partial(_agg_body, npass, nslab, nb))


_BR = 1000


def _scale_body(p_ref, x_ref, y_ref, dis_ref):
    d = 1.0 + p_ref[0][:, 0:1] + p_ref[1][:, 0:1]
    iv = lax.rsqrt(d)
    dis_ref[...] = iv
    y_ref[...] = x_ref[...] * iv


def _scale_call(degp, x):
    return pl.pallas_call(
        _scale_body,
        grid=(N // _BR,),
        in_specs=[
            pl.BlockSpec((NC, _BR, 128), lambda i: (0, i, 0)),
            pl.BlockSpec((_BR, x.shape[1]), lambda i: (i, 0)),
        ],
        out_specs=[
            pl.BlockSpec((_BR, x.shape[1]), lambda i: (i, 0)),
            pl.BlockSpec((_BR, 1), lambda i: (i, 0)),
        ],
        out_shape=[
            jax.ShapeDtypeStruct((N, x.shape[1]), jnp.float32),
            jax.ShapeDtypeStruct((N, 1), jnp.float32),
        ],
    )(degp, x)


_BN_S = float(1.0 / np.sqrt(1.0 + 1e-5))


def _mm1_body(z_ref, y_ref, dis_ref, w_ref, b_ref, g_ref, be_ref, y2_ref):
    zcat = jnp.concatenate([z_ref[k] for k in range(2)], axis=1)
    u = (zcat + y_ref[...]) * dis_ref[...]
    h = jnp.dot(u, w_ref[...], preferred_element_type=jnp.float32) + b_ref[...]
    h = h * (g_ref[...] * _BN_S) + be_ref[...]
    h = jnp.maximum(h, 0.0)
    y2_ref[...] = h * dis_ref[...]


def _mm1_call(z1, y1, dis, W1, b1, g1, be1):
    return pl.pallas_call(
        _mm1_body,
        grid=(N // _BR,),
        in_specs=[
            pl.BlockSpec((2, _BR, 128), lambda i: (0, i, 0)),
            pl.BlockSpec((_BR, 256), lambda i: (i, 0)),
            pl.BlockSpec((_BR, 1), lambda i: (i, 0)),
            pl.BlockSpec((256, 512), lambda i: (0, 0)),
            pl.BlockSpec((1, 512), lambda i: (0, 0)),
            pl.BlockSpec((1, 512), lambda i: (0, 0)),
            pl.BlockSpec((1, 512), lambda i: (0, 0)),
        ],
        out_specs=pl.BlockSpec((_BR, 512), lambda i: (i, 0)),
        out_shape=jax.ShapeDtypeStruct((N, 512), jnp.float32),
    )(z1, y1, dis, W1, b1, g1, be1)


def _mm2_body(z_ref, y_ref, dis_ref, w_ref, b_ref, g_ref, be_ref, fw_ref, fb_ref,
              o_ref):
    zcat = jnp.concatenate([z_ref[k] for k in range(4)], axis=1)
    u = (zcat + y_ref[...]) * dis_ref[...]
    h = jnp.dot(u, w_ref[...], preferred_element_type=jnp.float32) + b_ref[...]
    h = h * (g_ref[...] * _BN_S) + be_ref[...]
    h = jnp.maximum(h, 0.0)
    o = jnp.dot(h, fw_ref[...], preferred_element_type=jnp.float32) + fb_ref[...]
    o_ref[...] = jax.nn.sigmoid(o)


def _mm2_call(z2, y2, dis, W2, b2, g2, be2, fcW, fcb):
    return pl.pallas_call(
        _mm2_body,
        grid=(N // _BR,),
        in_specs=[
            pl.BlockSpec((4, _BR, 128), lambda i: (0, i, 0)),
            pl.BlockSpec((_BR, 512), lambda i: (i, 0)),
            pl.BlockSpec((_BR, 1), lambda i: (i, 0)),
            pl.BlockSpec((512, 512), lambda i: (0, 0)),
            pl.BlockSpec((1, 512), lambda i: (0, 0)),
            pl.BlockSpec((1, 512), lambda i: (0, 0)),
            pl.BlockSpec((1, 512), lambda i: (0, 0)),
            pl.BlockSpec((512, 256), lambda i: (0, 0)),
            pl.BlockSpec((1, 256), lambda i: (0, 0)),
        ],
        out_specs=pl.BlockSpec((_BR, 256), lambda i: (i, 0)),
        out_shape=jax.ShapeDtypeStruct((N, 256), jnp.float32),
    )(z2, y2, dis, W2, b2, g2, be2, fcW, fcb)


def kernel(x, edge_index, W1, b1, gamma1, beta1, W2, b2, gamma2, beta2, fcW, fcb):
    E = edge_index.shape[1]
    quantum = EB * NS * NC * NBUF
    epad = ((E + quantum - 1) // quantum) * quantum
    src = edge_index[0].astype(jnp.int32)
    dst = edge_index[1].astype(jnp.int32)
    src2 = jnp.concatenate([src, jnp.zeros((epad - E,), jnp.int32)]).reshape(-1, EB)
    dst2 = jnp.concatenate([dst, jnp.full((epad - E,), N, jnp.int32)]).reshape(-1, EB)

    zeros_w = jnp.zeros((STRIPE, 128), jnp.float32)
    ones_w = jnp.ones((EB, 128), jnp.float32)

    nb = epad // EB // NS
    degp = _make_deg_kernel()(dst2, zeros_w, ones_w)
    y1, dis = _scale_call(degp, x)
    z1 = _make_agg_kernel(1, nb)(y1.reshape(2 * N, 128), src2, dst2, zeros_w)
    y2 = _mm1_call(z1, y1, dis, W1, b1.reshape(1, -1), gamma1.reshape(1, -1),
                   beta1.reshape(1, -1))
    z2 = _make_agg_kernel(2, nb)(y2.reshape(4 * N, 128), src2, dst2, zeros_w)
    out = _mm2_call(z2, y2, dis, W2, b2.reshape(1, -1), gamma2.reshape(1, -1),
                    beta2.reshape(1, -1), fcW, fcb.reshape(1, -1))
    return out
